# baseline (speedup 1.0000x reference)
import jax
import jax.numpy as jnp
from jax.experimental import pallas as pl

_BB = 8
_T = _BB * 1024


def _vq_block(z_ref, w_ref, w2_ref, zsq_ref, wsq_ref, zq_ref, idx_ref):
    w = w_ref[...]
    w2 = w2_ref[...]
    K = w.shape[0]
    wsq = wsq_ref[...]
    s2s = [jax.lax.dot_general(jnp.transpose(z_ref[b], (1, 0)), w2,
                               (((1,), (1,)), ((), ())),
                               preferred_element_type=jnp.float32)
           for b in range(_BB)]
    ds = [(zsq_ref[pl.ds(b * 1024, 1024), :] + wsq) - s2s[b]
          for b in range(_BB)]
    ms = [jnp.min(ds[b], axis=1, keepdims=True) for b in range(_BB)]
    iota = jax.lax.broadcasted_iota(jnp.int32, ds[0].shape, 1)
    idxs = [jnp.min(jnp.where(ds[b] == ms[b], iota, K), axis=1, keepdims=True)
            for b in range(_BB)]
    for b in range(_BB):
        idx_ref[pl.ds(b * 1024, 1024), :] = idxs[b]
    zqs = [jnp.dot((iota == idxs[b]).astype(jnp.float32), w,
                   preferred_element_type=jnp.float32) for b in range(_BB)]
    for b in range(_BB):
        zq_ref[b] = jnp.transpose(zqs[b], (1, 0))


def kernel(z, W):
    B, C, H, Wd = z.shape
    HW = H * Wd
    N = B * HW
    K = W.shape[0]
    z3 = z.reshape(B, C, HW)
    zsq = jnp.sum(jnp.transpose(z3, (0, 2, 1)).reshape(N, C) ** 2,
                  axis=1, keepdims=True)
    wsq = jnp.sum(W ** 2, axis=1).reshape(1, K)
    zq3, idx = pl.pallas_call(
        _vq_block,
        grid=(B // _BB,),
        in_specs=[pl.BlockSpec((_BB, C, HW), lambda i: (i, 0, 0)),
                  pl.BlockSpec((K, C), lambda i: (0, 0)),
                  pl.BlockSpec((K, C), lambda i: (0, 0)),
                  pl.BlockSpec((_T, 1), lambda i: (i, 0)),
                  pl.BlockSpec((1, K), lambda i: (0, 0))],
        out_specs=[pl.BlockSpec((_BB, C, HW), lambda i: (i, 0, 0)),
                   pl.BlockSpec((_T, 1), lambda i: (i, 0))],
        out_shape=[jax.ShapeDtypeStruct((B, C, HW), jnp.float32),
                   jax.ShapeDtypeStruct((N, 1), jnp.int32)],
    )(z3, W, W + W, zsq, wsq)
    return zq3.reshape(B, C, H, Wd), idx.reshape(N)


# speedup vs baseline: 1.0099x; 1.0099x over previous
import jax
import jax.numpy as jnp
from jax.experimental import pallas as pl

_BB = 4
_T = _BB * 1024


def _vq_block(z_ref, w_ref, w2_ref, zsq_ref, wsq_ref, zq_ref, idx_ref):
    w = w_ref[...]
    w2 = w2_ref[...]
    K = w.shape[0]
    wsq = wsq_ref[...]
    s2s = [jax.lax.dot_general(jnp.transpose(z_ref[b], (1, 0)), w2,
                               (((1,), (1,)), ((), ())),
                               preferred_element_type=jnp.float32)
           for b in range(_BB)]
    ds = [(zsq_ref[pl.ds(b * 1024, 1024), :] + wsq) - s2s[b]
          for b in range(_BB)]
    ms = [jnp.min(ds[b], axis=1, keepdims=True) for b in range(_BB)]
    iota = jax.lax.broadcasted_iota(jnp.int32, ds[0].shape, 1)
    idxs = [jnp.min(jnp.where(ds[b] == ms[b], iota, K), axis=1, keepdims=True)
            for b in range(_BB)]
    for b in range(_BB):
        idx_ref[pl.ds(b * 1024, 1024), :] = idxs[b]
    zqs = [jnp.dot((iota == idxs[b]).astype(jnp.float32), w,
                   preferred_element_type=jnp.float32) for b in range(_BB)]
    for b in range(_BB):
        zq_ref[b] = jnp.transpose(zqs[b], (1, 0))


def kernel(z, W):
    B, C, H, Wd = z.shape
    HW = H * Wd
    N = B * HW
    K = W.shape[0]
    z3 = z.reshape(B, C, HW)
    zsq = jnp.sum(jnp.transpose(z3, (0, 2, 1)).reshape(N, C) ** 2,
                  axis=1, keepdims=True)
    wsq = jnp.sum(W ** 2, axis=1).reshape(1, K)
    zq3, idx = pl.pallas_call(
        _vq_block,
        grid=(B // _BB,),
        in_specs=[pl.BlockSpec((_BB, C, HW), lambda i: (i, 0, 0)),
                  pl.BlockSpec((K, C), lambda i: (0, 0)),
                  pl.BlockSpec((K, C), lambda i: (0, 0)),
                  pl.BlockSpec((_T, 1), lambda i: (i, 0)),
                  pl.BlockSpec((1, K), lambda i: (0, 0))],
        out_specs=[pl.BlockSpec((_BB, C, HW), lambda i: (i, 0, 0)),
                   pl.BlockSpec((_T, 1), lambda i: (i, 0))],
        out_shape=[jax.ShapeDtypeStruct((B, C, HW), jnp.float32),
                   jax.ShapeDtypeStruct((N, 1), jnp.int32)],
    )(z3, W, W + W, zsq, wsq)
    return zq3.reshape(B, C, H, Wd), idx.reshape(N)


# speedup vs baseline: 1.1396x; 1.1284x over previous
import jax
import jax.numpy as jnp
from jax.experimental import pallas as pl

_BB = 4
_T = _BB * 1024


def _vq_block(z_ref, w_ref, w2_ref, zsq_ref, wsq_ref, zq_ref, idx_ref):
    w = w_ref[...]
    w2 = w2_ref[...]
    K = w.shape[0]
    wsq = wsq_ref[...]
    zsqs = [jnp.transpose(zsq_ref[0, :, pl.ds(b * 1024, 1024)], (1, 0))
            for b in range(_BB)]
    s2s = [jax.lax.dot_general(jnp.transpose(z_ref[b], (1, 0)), w2,
                               (((1,), (1,)), ((), ())),
                               preferred_element_type=jnp.float32)
           for b in range(_BB)]
    ds = [(zsqs[b] + wsq) - s2s[b] for b in range(_BB)]
    ms = [jnp.min(ds[b], axis=1, keepdims=True) for b in range(_BB)]
    iota = jax.lax.broadcasted_iota(jnp.int32, ds[0].shape, 1)
    idxs = [jnp.min(jnp.where(ds[b] == ms[b], iota, K), axis=1, keepdims=True)
            for b in range(_BB)]
    for b in range(_BB):
        idx_ref[0, :, pl.ds(b * 1024, 1024)] = jnp.transpose(idxs[b], (1, 0))
    zqs = [jnp.dot((iota == idxs[b]).astype(jnp.float32), w,
                   preferred_element_type=jnp.float32) for b in range(_BB)]
    for b in range(_BB):
        zq_ref[b] = jnp.transpose(zqs[b], (1, 0))


def kernel(z, W):
    B, C, H, Wd = z.shape
    HW = H * Wd
    N = B * HW
    K = W.shape[0]
    NB = B // _BB
    z3 = z.reshape(B, C, HW)
    zsq = jnp.sum(jnp.transpose(z3, (0, 2, 1)).reshape(N, C) ** 2,
                  axis=1).reshape(NB, 1, _T)
    wsq = jnp.sum(W ** 2, axis=1).reshape(1, K)
    zq3, idx = pl.pallas_call(
        _vq_block,
        grid=(NB,),
        in_specs=[pl.BlockSpec((_BB, C, HW), lambda i: (i, 0, 0)),
                  pl.BlockSpec((K, C), lambda i: (0, 0)),
                  pl.BlockSpec((K, C), lambda i: (0, 0)),
                  pl.BlockSpec((1, 1, _T), lambda i: (i, 0, 0)),
                  pl.BlockSpec((1, K), lambda i: (0, 0))],
        out_specs=[pl.BlockSpec((_BB, C, HW), lambda i: (i, 0, 0)),
                   pl.BlockSpec((1, 1, _T), lambda i: (i, 0, 0))],
        out_shape=[jax.ShapeDtypeStruct((B, C, HW), jnp.float32),
                   jax.ShapeDtypeStruct((NB, 1, _T), jnp.int32)],
    )(z3, W, W + W, zsq, wsq)
    return zq3.reshape(B, C, H, Wd), idx.reshape(N)
